# baseline (speedup 1.0000x reference)
import functools

import jax
import jax.numpy as jnp
import numpy as np
from jax import lax
from jax.experimental import pallas as pl
from jax.experimental.pallas import tpu as pltpu
from jax.experimental.pallas import tpu_sc as plsc

_D = 128
_L = 200
_SCALE = float(np.sqrt(_D))
_LANES = 16

_NC = 2
_NS = 16
_NW = _NC * _NS

_G = 100
_NBUF = 2


def _make_pe(seq_len, d):
    pos = np.arange(seq_len)[:, None].astype(np.float64)
    i = np.arange(d)[None, :].astype(np.float64)
    angle = pos / np.power(10000.0, 2.0 * np.floor(i / 2.0) / d)
    pe = np.zeros((seq_len, d), dtype=np.float64)
    pe[:, 0::2] = np.sin(angle[:, 0::2])
    pe[:, 1::2] = np.cos(angle[:, 1::2])
    return pe.astype(np.float32)


_PE = _make_pe(_L, _D)


@functools.partial(jax.jit, static_argnames=("nb", "nl"))
def _sc_gather_pe(table, idxa, idxb, pe, nb, nl):
    n = nb * nl
    n_win = nb // _NW
    mesh = plsc.VectorSubcoreMesh(core_axis_name="c", subcore_axis_name="s")

    @functools.partial(
        pl.kernel,
        out_type=jax.ShapeDtypeStruct((n, _D), jnp.float32),
        mesh=mesh,
        scratch_types=[
            pltpu.VMEM((_NBUF, 1, 104), jnp.int32),
            pltpu.VMEM((_NBUF, 1, 96), jnp.int32),
            pltpu.VMEM((_L, _D), jnp.float32),
            pltpu.VMEM((_L, _D), jnp.float32),
            pltpu.VMEM((_L, _D), jnp.float32),
            pltpu.VMEM((_L, _D), jnp.float32),
            pltpu.VMEM((_L, _D), jnp.float32),
            pltpu.SemaphoreType.DMA,
            pltpu.SemaphoreType.DMA,
            pltpu.SemaphoreType.DMA,
            pltpu.SemaphoreType.DMA,
            pltpu.SemaphoreType.DMA,
            pltpu.SemaphoreType.DMA,
            pltpu.SemaphoreType.DMA,
            pltpu.SemaphoreType.DMA,
            pltpu.SemaphoreType.DMA,
            pltpu.SemaphoreType.DMA,
        ],
    )
    def k(table_hbm, idxa_hbm, idxb_hbm, pe_hbm, out_hbm,
          idx_va, idx_vb, pe_v, g0, g1, o0, o1,
          is0, is1, gs00, gs01, gs10, gs11, os00, os01, os10, os11):
        grows = [g0, g1]
        orows = [o0, o1]
        isem = [is0, is1]
        gsem = [[gs00, gs01], [gs10, gs11]]
        osem = [[os00, os01], [os10, os11]]
        wid = lax.axis_index("s") * _NC + lax.axis_index("c")
        seq0 = wid * n_win

        pltpu.sync_copy(pe_hbm, pe_v)

        def start_idx(b, w):
            pltpu.async_copy(idxa_hbm.at[seq0 + w], idx_va.at[b], isem[b])
            pltpu.async_copy(idxb_hbm.at[seq0 + w], idx_vb.at[b], isem[b])

        def wait_idx(b, w):
            pltpu.make_async_copy(idxa_hbm.at[seq0 + w], idx_va.at[b],
                                  isem[b]).wait()
            pltpu.make_async_copy(idxb_hbm.at[seq0 + w], idx_vb.at[b],
                                  isem[b]).wait()

        _HOFF = (0, 104)
        _HLEN = (104, 96)

        _IDXV = None

        def start_gather(b, w, h):
            iv = idx_va if h == 0 else idx_vb
            pltpu.async_copy(
                table_hbm.at[iv.at[b, 0]],
                grows[b].at[pl.ds(_HOFF[h], _HLEN[h])], gsem[b][h])

        def wait_gather(b, w, h):
            iv = idx_va if h == 0 else idx_vb
            pltpu.make_async_copy(
                table_hbm.at[iv.at[b, 0]],
                grows[b].at[pl.ds(_HOFF[h], _HLEN[h])], gsem[b][h]).wait()

        def start_out(b, w, h):
            pltpu.async_copy(
                orows[b].at[pl.ds(_HOFF[h], _HLEN[h])],
                out_hbm.at[pl.ds((seq0 + w) * _L + _HOFF[h], _HLEN[h])],
                osem[b][h])

        def wait_out(b, w, h):
            pltpu.make_async_copy(
                orows[b].at[pl.ds(_HOFF[h], _HLEN[h])],
                out_hbm.at[pl.ds((seq0 + w) * _L + _HOFF[h], _HLEN[h])],
                osem[b][h]).wait()

        for b in range(_NBUF):
            start_idx(b, b)
            wait_idx(b, b)
            for h in range(2):
                start_gather(b, b, h)

        @pl.loop(0, n_win, step=_NBUF)
        def _win(w0):
            for b in range(_NBUF):
                w = w0 + b

                @pl.when(w + _NBUF < n_win)
                def _():
                    start_idx(b, w + _NBUF)

                g = grows[b]
                o = orows[b]

                for h in range(2):
                    wait_gather(b, w, h)

                    @pl.when(w >= _NBUF)
                    def _():
                        wait_out(b, w - _NBUF, h)

                    @pl.loop(_HOFF[h], _HOFF[h] + _HLEN[h])
                    def _row(r):
                        for c in range(_D // _LANES):
                            sl = pl.ds(c * _LANES, _LANES)
                            o[r, sl] = g[r, sl] * _SCALE + pe_v[r, sl]

                    @pl.when(w + _NBUF < n_win)
                    def _():
                        if h == 0:
                            wait_idx(b, w + _NBUF)
                        start_gather(b, w + _NBUF, h)

                    start_out(b, w, h)

        for b in range(_NBUF):
            for h in range(2):
                wait_out(b, n_win - _NBUF + b, h)

    return k(table, idxa, idxb, pe)


def kernel(inputs, table, training):
    b, l = inputs.shape
    idxa = inputs[:, :104].reshape(b, 1, 104)
    idxb = inputs[:, 104:].reshape(b, 1, 96)
    out = _sc_gather_pe(table, idxa, idxb, _PE, b, l)
    return out.reshape(b, l, _D)


# speedup vs baseline: 1.0098x; 1.0098x over previous
import functools

import jax
import jax.numpy as jnp
import numpy as np
from jax import lax
from jax.experimental import pallas as pl
from jax.experimental.pallas import tpu as pltpu
from jax.experimental.pallas import tpu_sc as plsc

_D = 128
_L = 200
_SCALE = float(np.sqrt(_D))
_LANES = 16

_NC = 2
_NS = 16
_NW = _NC * _NS

_G = 100
_NBUF = 2


def _make_pe(seq_len, d):
    pos = np.arange(seq_len)[:, None].astype(np.float64)
    i = np.arange(d)[None, :].astype(np.float64)
    angle = pos / np.power(10000.0, 2.0 * np.floor(i / 2.0) / d)
    pe = np.zeros((seq_len, d), dtype=np.float64)
    pe[:, 0::2] = np.sin(angle[:, 0::2])
    pe[:, 1::2] = np.cos(angle[:, 1::2])
    return pe.astype(np.float32)


_PE = _make_pe(_L, _D)


@functools.partial(jax.jit, static_argnames=("nb", "nl"))
def _sc_gather_pe(table, idx3d, pe, nb, nl):
    n = nb * nl
    n_win = nb // _NW
    mesh = plsc.VectorSubcoreMesh(core_axis_name="c", subcore_axis_name="s")

    @functools.partial(
        pl.kernel,
        out_type=jax.ShapeDtypeStruct((n, _D), jnp.float32),
        mesh=mesh,
        scratch_types=[
            pltpu.VMEM((_NBUF, 2, _G), jnp.int32),
            pltpu.VMEM((_L, _D), jnp.float32),
            pltpu.VMEM((_L, _D), jnp.float32),
            pltpu.VMEM((_L, _D), jnp.float32),
            pltpu.VMEM((_L, _D), jnp.float32),
            pltpu.VMEM((_L, _D), jnp.float32),
            pltpu.SemaphoreType.DMA,
            pltpu.SemaphoreType.DMA,
            pltpu.SemaphoreType.DMA,
            pltpu.SemaphoreType.DMA,
            pltpu.SemaphoreType.DMA,
            pltpu.SemaphoreType.DMA,
        ],
    )
    def k(table_hbm, idx_hbm, pe_hbm, out_hbm,
          idx_v, pe_v, g0, g1, o0, o1, is0, is1, gs0, gs1, os0, os1):
        grows = [g0, g1]
        orows = [o0, o1]
        isem = [is0, is1]
        gsem = [gs0, gs1]
        osem = [os0, os1]
        wid = lax.axis_index("s") * _NC + lax.axis_index("c")
        seq0 = wid * n_win

        pltpu.sync_copy(pe_hbm, pe_v)

        def start_idx(b, w):
            pltpu.async_copy(idx_hbm.at[seq0 + w], idx_v.at[b], isem[b])

        def wait_idx(b, w):
            pltpu.make_async_copy(idx_hbm.at[seq0 + w], idx_v.at[b],
                                  isem[b]).wait()

        def start_gather(b, w):
            for h in range(2):
                pltpu.async_copy(table_hbm.at[idx_v.at[b, h]],
                                 grows[b].at[pl.ds(h * _G, _G)], gsem[b])

        def wait_gather(b, w):
            for h in range(2):
                pltpu.make_async_copy(table_hbm.at[idx_v.at[b, h]],
                                      grows[b].at[pl.ds(h * _G, _G)],
                                      gsem[b]).wait()

        _HOFF = (0, 104)
        _HLEN = (104, 96)

        def start_out(b, w, h):
            pltpu.async_copy(
                orows[b].at[pl.ds(_HOFF[h], _HLEN[h])],
                out_hbm.at[pl.ds((seq0 + w) * _L + _HOFF[h], _HLEN[h])],
                osem[b])

        def wait_out(b, w, h):
            pltpu.make_async_copy(
                orows[b].at[pl.ds(_HOFF[h], _HLEN[h])],
                out_hbm.at[pl.ds((seq0 + w) * _L + _HOFF[h], _HLEN[h])],
                osem[b]).wait()

        for b in range(_NBUF):
            start_idx(b, b)
            wait_idx(b, b)
            start_gather(b, b)

        @pl.loop(0, n_win, step=_NBUF)
        def _win(w0):
            for b in range(_NBUF):
                w = w0 + b
                wait_gather(b, w)

                @pl.when(w + _NBUF < n_win)
                def _():
                    start_idx(b, w + _NBUF)

                g = grows[b]
                o = orows[b]

                for h in range(2):
                    @pl.when(w >= _NBUF)
                    def _():
                        wait_out(b, w - _NBUF, h)

                    @pl.loop(_HOFF[h], _HOFF[h] + _HLEN[h])
                    def _row(r):
                        for c in range(_D // _LANES):
                            sl = pl.ds(c * _LANES, _LANES)
                            o[r, sl] = g[r, sl] * _SCALE + pe_v[r, sl]

                    start_out(b, w, h)

                @pl.when(w + _NBUF < n_win)
                def _():
                    wait_idx(b, w + _NBUF)
                    start_gather(b, w + _NBUF)

        for b in range(_NBUF):
            for h in range(2):
                wait_out(b, n_win - _NBUF + b, h)

    return k(table, idx3d, pe)


def kernel(inputs, table, training):
    b, l = inputs.shape
    idx3d = inputs.reshape(b, 2, _G)
    out = _sc_gather_pe(table, idx3d, _PE, b, l)
    return out.reshape(b, l, _D)
